# baseline (speedup 1.0000x reference)
import jax
import jax.numpy as jnp
from jax import lax
from jax.experimental import pallas as pl
from jax.experimental.pallas import tpu as pltpu
from jax.experimental.pallas import tpu_sc as plsc

_N_REAL = 50000
_N_PAD = 51200
_H = 16
_NC = 2
_NS = 16
_ROWS_PER_TILE = _N_PAD // _NS
_RCHUNK = 320
_NCHUNK = _ROWS_PER_TILE // _RCHUNK
_E_PAD = 1_638_400
_STREAM = 256
_IDX_BLKS = _E_PAD // _STREAM
_GROUPS = 4
_SB_PER_TILE = _IDX_BLKS // _NS // _GROUPS
_DEG_ROUNDS = 2
_DEG_RANGE = _N_PAD // _DEG_ROUNDS
_DEG_ROWS = _DEG_RANGE // 16
_DEG_TROWS = _DEG_ROWS // _NS
_EPS = 1e-07
_N_LAYERS = 3

_GDN = lax.GatherDimensionNumbers(
    offset_dims=(), collapsed_slice_dims=(0,), start_index_map=(0,))
_LANES = None


def _vgather(v, idx):
    return lax.gather(v, idx[:, None], _GDN, (1,),
                      mode=lax.GatherScatterMode.PROMISE_IN_BOUNDS)


def _rsqrt_newton(a):
    i = lax.bitcast_convert_type(a, jnp.int32)
    i = jnp.int32(0x5F3759DF) - lax.shift_right_arithmetic(i, jnp.int32(1))
    r = lax.bitcast_convert_type(i, jnp.float32)
    half = a * 0.5
    for _ in range(3):
        r = r * (1.5 - half * r * r)
    return r


def _body(xs, rows, cols_plain, cols, out, y_hbm, c2_hbm, acc_sh, degp_sh,
          rbuf, cbuf, gring, abuf, gbuf, c2buf, sbuf, zerosb,
          degacc, sumtmp, redtmp, degw, gsem, ssem):
    ci = lax.axis_index("c")
    tid = lax.axis_index("s")
    node_base = tid * _ROWS_PER_TILE
    blk_base = tid * (_SB_PER_TILE * _GROUPS)

    ones16 = jnp.ones((16,), jnp.float32)
    zeros16 = jnp.zeros((16,), jnp.float32)

    def fill_zeros(i, _):
        zerosb[i, :] = zeros16
        return 0

    lax.fori_loop(0, _RCHUNK, fill_zeros, 0)

    def zero_chunk(c, _):
        off = node_base + c * _RCHUNK
        pltpu.sync_copy(zerosb, acc_sh.at[pl.ds(off, _RCHUNK)])
        return 0

    lax.fori_loop(0, _NCHUNK, zero_chunk, 0)
    plsc.subcore_barrier()

    for rnd in range(_DEG_ROUNDS):
        lo = jnp.int32(rnd * _DEG_RANGE)
        hi = jnp.int32((rnd + 1) * _DEG_RANGE)

        def zdeg(j, _):
            degacc[j, :] = zeros16
            return 0

        lax.fori_loop(0, _DEG_ROWS, zdeg, 0)

        def count_sb(sb, _):
            blk_off = blk_base + sb * _GROUPS
            pltpu.sync_copy(rows.at[pl.ds(blk_off, _GROUPS)], rbuf)
            pltpu.sync_copy(cols_plain.at[pl.ds(blk_off, _GROUPS)], cbuf)

            def count_vec(i, _):
                g = i // 16
                o = (i % 16) * 16
                for buf in (rbuf, cbuf):
                    v = buf[g, pl.ds(o, 16)]
                    m = (v >= lo) & (v < hi)
                    local = jnp.where(m, v - lo, 0)
                    plsc.addupdate_scatter(
                        degacc,
                        [lax.shift_right_logical(local, 4), local & 15],
                        ones16, mask=m)
                return 0

            lax.fori_loop(0, _GROUPS * _STREAM // 16, count_vec, 0)
            return 0

        lax.fori_loop(0, _SB_PER_TILE, count_sb, 0)
        pltpu.sync_copy(degacc, acc_sh.at[pl.ds(tid * _DEG_ROWS, _DEG_ROWS)])
        plsc.subcore_barrier()

        def zsum(r, _):
            sumtmp[r, :] = zeros16
            return 0

        lax.fori_loop(0, _DEG_TROWS, zsum, 0)

        def red_partial(p, _):
            pltpu.sync_copy(
                acc_sh.at[pl.ds(p * _DEG_ROWS + tid * _DEG_TROWS,
                                _DEG_TROWS)], redtmp)

            def addrow(r, _):
                sumtmp[r, :] = sumtmp[r, :] + redtmp[r, :]
                return 0

            lax.fori_loop(0, _DEG_TROWS, addrow, 0)
            return 0

        lax.fori_loop(0, _NS, red_partial, 0)
        pltpu.sync_copy(
            sumtmp,
            degp_sh.at[pl.ds(rnd * _DEG_ROWS + tid * _DEG_TROWS,
                             _DEG_TROWS)])
        plsc.subcore_barrier()

    lax.fori_loop(0, _NCHUNK, zero_chunk, 0)
    plsc.subcore_barrier()

    lanes = [jnp.full((16,), j, jnp.int32) for j in range(16)]

    def init_chunk(c, _):
        off = node_base + c * _RCHUNK
        pltpu.sync_copy(
            degp_sh.at[pl.ds(off // 16, _RCHUNK // 16)], degw)
        pltpu.sync_copy(xs.at[ci, pl.ds(off, _RCHUNK)], gbuf)

        def init_grp(g, _):
            d = degw[g, :] + _EPS
            c2 = 1.0 / d
            cc = c2 * _rsqrt_newton(c2)
            for j in range(16):
                r = g * 16 + j
                y0 = gbuf[r, :] * _vgather(cc, lanes[j])
                c2buf[r, :] = _vgather(c2, lanes[j])
                sbuf[r, :] = y0
                abuf[r, :] = y0
            return 0

        lax.fori_loop(0, _RCHUNK // 16, init_grp, 0)
        pltpu.sync_copy(c2buf, c2_hbm.at[ci, pl.ds(off, _RCHUNK)])
        pltpu.sync_copy(sbuf, out.at[ci, pl.ds(off, _RCHUNK)])
        pltpu.sync_copy(abuf, y_hbm.at[pl.ds(ci * _N_PAD + off, _RCHUNK)])
        return 0

    lax.fori_loop(0, _NCHUNK, init_chunk, 0)
    plsc.subcore_barrier()

    def gather_grp(grp, half):
        return (y_hbm.at[cbuf.at[grp]], gring.at[half], gsem)

    def scatter_grp(grp, half):
        return (gring.at[half], acc_sh.at[rbuf.at[grp]], ssem)

    def edge_sb(sb, _):
        blk_off = blk_base + sb * _GROUPS
        pltpu.sync_copy(rows.at[pl.ds(blk_off, _GROUPS)], rbuf)
        pltpu.sync_copy(cols.at[ci, pl.ds(blk_off, _GROUPS)], cbuf)

        def fire_g(grp):
            pltpu.async_copy(*gather_grp(grp, lax.rem(grp, 2)))

        def drain_g(grp):
            pltpu.make_async_copy(*gather_grp(grp, lax.rem(grp, 2))).wait()

        def fire_s(grp):
            pltpu.async_copy(*scatter_grp(grp, lax.rem(grp, 2)), add=True)

        def drain_s(grp):
            pltpu.make_async_copy(*scatter_grp(grp, lax.rem(grp, 2))).wait()

        fire_g(jnp.int32(0))

        def grp_body(grp, _):
            drain_g(grp)

            @pl.when(grp > 0)
            def _():
                drain_s(grp - 1)

            fire_s(grp)

            @pl.when(grp < _GROUPS - 1)
            def _():
                fire_g(grp + 1)

            return 0

        lax.fori_loop(0, _GROUPS, grp_body, 0)
        drain_s(jnp.int32(_GROUPS - 1))
        return 0

    for layer in range(_N_LAYERS):
        last = layer == _N_LAYERS - 1

        lax.fori_loop(0, _SB_PER_TILE, edge_sb, 0)
        plsc.subcore_barrier()

        def rescale_chunk(c, _):
            off = node_base + c * _RCHUNK
            pltpu.sync_copy(acc_sh.at[pl.ds(off, _RCHUNK)], abuf)
            pltpu.sync_copy(c2_hbm.at[ci, pl.ds(off, _RCHUNK)], c2buf)
            pltpu.sync_copy(out.at[ci, pl.ds(off, _RCHUNK)], sbuf)
            if not last:
                pltpu.sync_copy(zerosb, acc_sh.at[pl.ds(off, _RCHUNK)])

            def rescale_row(r, _):
                c2 = c2buf[r, :]
                val = abuf[r, :] * c2
                s = sbuf[r, :] + val
                if last:
                    sbuf[r, :] = s * _rsqrt_newton(c2) * 0.25
                else:
                    sbuf[r, :] = s
                    abuf[r, :] = val
                return 0

            lax.fori_loop(0, _RCHUNK, rescale_row, 0)
            pltpu.sync_copy(sbuf, out.at[ci, pl.ds(off, _RCHUNK)])
            if not last:
                pltpu.sync_copy(
                    abuf, y_hbm.at[pl.ds(ci * _N_PAD + off, _RCHUNK)])
            return 0

        lax.fori_loop(0, _NCHUNK, rescale_chunk, 0)
        if not last:
            plsc.subcore_barrier()


@jax.jit
def _lightgcn(xs, rows, cols_plain, cols):
    mesh = plsc.VectorSubcoreMesh(core_axis_name="c", subcore_axis_name="s")
    out, _, _ = pl.kernel(
        _body,
        out_type=(
            jax.ShapeDtypeStruct((_NC, _N_PAD, _H), jnp.float32),
            jax.ShapeDtypeStruct((_NC * _N_PAD, _H), jnp.float32),
            jax.ShapeDtypeStruct((_NC, _N_PAD, _H), jnp.float32),
        ),
        mesh=mesh,
        compiler_params=pltpu.CompilerParams(use_tc_tiling_on_sc=False,
                                             needs_layout_passes=False),
        scratch_types=[
            pltpu.VMEM_SHARED((_N_PAD, _H), jnp.float32),
            pltpu.VMEM_SHARED((_N_PAD // 16, 16), jnp.float32),
            pltpu.VMEM((_GROUPS, _STREAM), jnp.int32),
            pltpu.VMEM((_GROUPS, _STREAM), jnp.int32),
            pltpu.VMEM((2, _STREAM, _H), jnp.float32),
            pltpu.VMEM((_RCHUNK, _H), jnp.float32),
            pltpu.VMEM((_RCHUNK, _H), jnp.float32),
            pltpu.VMEM((_RCHUNK, _H), jnp.float32),
            pltpu.VMEM((_RCHUNK, _H), jnp.float32),
            pltpu.VMEM((_RCHUNK, _H), jnp.float32),
            pltpu.VMEM((_DEG_ROWS, 16), jnp.float32),
            pltpu.VMEM((_DEG_TROWS, 16), jnp.float32),
            pltpu.VMEM((_DEG_TROWS, 16), jnp.float32),
            pltpu.VMEM((_RCHUNK // 16, 16), jnp.float32),
            pltpu.SemaphoreType.DMA,
            pltpu.SemaphoreType.DMA,
        ],
    )(xs, rows, cols_plain, cols)
    return out


def kernel(user_emb, item_emb, edge_index):
    n_users = user_emb.shape[0]
    n_items = item_emb.shape[0]
    ego = jnp.concatenate([user_emb, item_emb], axis=0)
    ego = jnp.pad(ego, ((0, _N_PAD - _N_REAL), (0, 0)))
    xs = ego.reshape(_N_PAD, _NC, _H).transpose(1, 0, 2)

    n_edges = edge_index.shape[1]
    pad = _E_PAD - n_edges
    dummy = jnp.full((pad,), _N_REAL, jnp.int32)
    rows = jnp.concatenate([edge_index[0], dummy]).reshape(_IDX_BLKS, _STREAM)
    cols_plain = jnp.concatenate([edge_index[1], dummy]).reshape(
        _IDX_BLKS, _STREAM)
    cols = jnp.stack([cols_plain, cols_plain + _N_PAD])

    out = _lightgcn(xs, rows, cols_plain, cols)
    full = out.transpose(1, 0, 2).reshape(_N_PAD, _NC * _H)
    return (full[:n_users], full[n_users:n_users + n_items])


# speedup vs baseline: 1.4828x; 1.4828x over previous
import jax
import jax.numpy as jnp
from jax import lax
from jax.experimental import pallas as pl
from jax.experimental.pallas import tpu as pltpu
from jax.experimental.pallas import tpu_sc as plsc

_N_REAL = 50000
_N_PAD = 51200
_H = 16
_NC = 2
_NS = 16
_ROWS_PER_TILE = _N_PAD // _NS
_RCHUNK = 320
_NCHUNK = _ROWS_PER_TILE // _RCHUNK
_E_PAD = 1_638_400
_STREAM = 1024
_IDX_BLKS = _E_PAD // _STREAM
_GROUPS = 4
_SB_PER_TILE = _IDX_BLKS // _NS // _GROUPS
_NG = _SB_PER_TILE * _GROUPS
_EPS = 1e-07
_N_LAYERS = 3


def _rsqrt_newton(a):
    i = lax.bitcast_convert_type(a, jnp.int32)
    i = jnp.int32(0x5F3759DF) - lax.shift_right_arithmetic(i, jnp.int32(1))
    r = lax.bitcast_convert_type(i, jnp.float32)
    half = a * 0.5
    for _ in range(3):
        r = r * (1.5 - half * r * r)
    return r


def _body(xs, rows, cols_plain, cols, out, y_hbm, c2_hbm, acc_sh,
          ribuf, cibuf, gring, abuf, gbuf, c2buf, sbuf, zerosb,
          gsem, ssem, isem):
    ci = lax.axis_index("c")
    tid = lax.axis_index("s")
    node_base = tid * _ROWS_PER_TILE
    blk_base = tid * _NG

    ones16 = jnp.ones((16,), jnp.float32)
    zeros16 = jnp.zeros((16,), jnp.float32)

    onesb = gring.at[0]

    def fill_ones(i, _):
        onesb[i, :] = ones16
        return 0

    def fill_zeros(i, _):
        zerosb[i, :] = zeros16
        return 0

    lax.fori_loop(0, _STREAM, fill_ones, 0)
    lax.fori_loop(0, _RCHUNK, fill_zeros, 0)

    def zero_chunk(c, _):
        off = node_base + c * _RCHUNK
        pltpu.sync_copy(zerosb, acc_sh.at[pl.ds(off, _RCHUNK)])
        return 0

    lax.fori_loop(0, _NCHUNK, zero_chunk, 0)
    plsc.subcore_barrier()

    def deg_sb(sb, _):
        blk_off = blk_base + sb * _GROUPS
        pltpu.sync_copy(rows.at[pl.ds(blk_off, _GROUPS)], ribuf.at[0])
        pltpu.sync_copy(cols_plain.at[pl.ds(blk_off, _GROUPS)], cibuf.at[0])

        def fire(grp, _):
            pltpu.async_copy(onesb, acc_sh.at[ribuf.at[0, grp]], ssem,
                             add=True)
            pltpu.async_copy(onesb, acc_sh.at[cibuf.at[0, grp]], ssem,
                             add=True)
            return 0

        def drain(grp, _):
            pltpu.make_async_copy(
                onesb, acc_sh.at[ribuf.at[0, grp]], ssem).wait()
            pltpu.make_async_copy(
                onesb, acc_sh.at[cibuf.at[0, grp]], ssem).wait()
            return 0

        lax.fori_loop(0, _GROUPS, fire, 0)
        lax.fori_loop(0, _GROUPS, drain, 0)
        return 0

    lax.fori_loop(0, _SB_PER_TILE, deg_sb, 0)
    plsc.subcore_barrier()

    def init_chunk(c, _):
        off = node_base + c * _RCHUNK
        pltpu.sync_copy(acc_sh.at[pl.ds(off, _RCHUNK)], abuf)
        pltpu.sync_copy(xs.at[ci, pl.ds(off, _RCHUNK)], gbuf)
        pltpu.sync_copy(zerosb, acc_sh.at[pl.ds(off, _RCHUNK)])

        def init_row(r, _):
            d = abuf[r, :] + _EPS
            c2 = 1.0 / d
            cc = c2 * _rsqrt_newton(c2)
            y0 = gbuf[r, :] * cc
            c2buf[r, :] = c2
            sbuf[r, :] = y0
            abuf[r, :] = y0
            return 0

        lax.fori_loop(0, _RCHUNK, init_row, 0)
        pltpu.sync_copy(c2buf, c2_hbm.at[ci, pl.ds(off, _RCHUNK)])
        pltpu.sync_copy(sbuf, out.at[ci, pl.ds(off, _RCHUNK)])
        pltpu.sync_copy(abuf, y_hbm.at[pl.ds(ci * _N_PAD + off, _RCHUNK)])
        return 0

    lax.fori_loop(0, _NCHUNK, init_chunk, 0)
    plsc.subcore_barrier()

    def gather_g(g):
        ib = lax.rem(g // _GROUPS, 2)
        return (y_hbm.at[cibuf.at[ib, lax.rem(g, _GROUPS)]],
                gring.at[lax.rem(g, 2)], gsem)

    def scatter_g(g):
        ib = lax.rem(g // _GROUPS, 2)
        return (gring.at[lax.rem(g, 2)],
                acc_sh.at[ribuf.at[ib, lax.rem(g, _GROUPS)]], ssem)

    def fire_idx(sb):
        ib = lax.rem(sb, 2)
        blk_off = blk_base + sb * _GROUPS
        pltpu.async_copy(rows.at[pl.ds(blk_off, _GROUPS)], ribuf.at[ib],
                         isem)
        pltpu.async_copy(cols.at[ci, pl.ds(blk_off, _GROUPS)], cibuf.at[ib],
                         isem)

    def wait_idx(sb):
        ib = lax.rem(sb, 2)
        blk_off = blk_base + sb * _GROUPS
        pltpu.make_async_copy(rows.at[pl.ds(blk_off, _GROUPS)],
                              ribuf.at[ib], isem).wait()
        pltpu.make_async_copy(cols.at[ci, pl.ds(blk_off, _GROUPS)],
                              cibuf.at[ib], isem).wait()

    def edge_pass():
        fire_idx(jnp.int32(0))
        wait_idx(jnp.int32(0))
        fire_idx(jnp.int32(1))
        pltpu.async_copy(*gather_g(jnp.int32(0)))

        def g_body(g, _):
            pltpu.make_async_copy(*gather_g(g)).wait()

            @pl.when(g > 0)
            def _():
                pltpu.make_async_copy(*scatter_g(g - 1)).wait()

            pltpu.async_copy(*scatter_g(g), add=True)

            @pl.when((lax.rem(g, _GROUPS) == 0) & (g > 0)
                     & (g + _GROUPS < _NG))
            def _():
                fire_idx(g // _GROUPS + 1)

            @pl.when(g + 1 < _NG)
            def _():
                @pl.when(lax.rem(g + 1, _GROUPS) == 0)
                def _():
                    wait_idx((g + 1) // _GROUPS)

                pltpu.async_copy(*gather_g(g + 1))

            return 0

        lax.fori_loop(0, _NG, g_body, 0)
        pltpu.make_async_copy(*scatter_g(jnp.int32(_NG - 1))).wait()

    for layer in range(_N_LAYERS):
        last = layer == _N_LAYERS - 1

        edge_pass()
        plsc.subcore_barrier()

        def rescale_chunk(c, _):
            off = node_base + c * _RCHUNK
            pltpu.sync_copy(acc_sh.at[pl.ds(off, _RCHUNK)], abuf)
            pltpu.sync_copy(c2_hbm.at[ci, pl.ds(off, _RCHUNK)], c2buf)
            pltpu.sync_copy(out.at[ci, pl.ds(off, _RCHUNK)], sbuf)
            if not last:
                pltpu.sync_copy(zerosb, acc_sh.at[pl.ds(off, _RCHUNK)])

            def rescale_row(r, _):
                c2 = c2buf[r, :]
                val = abuf[r, :] * c2
                s = sbuf[r, :] + val
                if last:
                    sbuf[r, :] = s * _rsqrt_newton(c2) * 0.25
                else:
                    sbuf[r, :] = s
                    abuf[r, :] = val
                return 0

            lax.fori_loop(0, _RCHUNK, rescale_row, 0)
            pltpu.sync_copy(sbuf, out.at[ci, pl.ds(off, _RCHUNK)])
            if not last:
                pltpu.sync_copy(
                    abuf, y_hbm.at[pl.ds(ci * _N_PAD + off, _RCHUNK)])
            return 0

        lax.fori_loop(0, _NCHUNK, rescale_chunk, 0)
        if not last:
            plsc.subcore_barrier()


@jax.jit
def _lightgcn(xs, rows, cols_plain, cols):
    mesh = plsc.VectorSubcoreMesh(core_axis_name="c", subcore_axis_name="s")
    out, _, _ = pl.kernel(
        _body,
        out_type=(
            jax.ShapeDtypeStruct((_NC, _N_PAD, _H), jnp.float32),
            jax.ShapeDtypeStruct((_NC * _N_PAD, _H), jnp.float32),
            jax.ShapeDtypeStruct((_NC, _N_PAD, _H), jnp.float32),
        ),
        mesh=mesh,
        compiler_params=pltpu.CompilerParams(use_tc_tiling_on_sc=False),
        scratch_types=[
            pltpu.VMEM_SHARED((_N_PAD, _H), jnp.float32),
            pltpu.VMEM((2, _GROUPS, _STREAM), jnp.int32),
            pltpu.VMEM((2, _GROUPS, _STREAM), jnp.int32),
            pltpu.VMEM((2, _STREAM, _H), jnp.float32),
            pltpu.VMEM((_RCHUNK, _H), jnp.float32),
            pltpu.VMEM((_RCHUNK, _H), jnp.float32),
            pltpu.VMEM((_RCHUNK, _H), jnp.float32),
            pltpu.VMEM((_RCHUNK, _H), jnp.float32),
            pltpu.VMEM((_RCHUNK, _H), jnp.float32),
            pltpu.SemaphoreType.DMA,
            pltpu.SemaphoreType.DMA,
            pltpu.SemaphoreType.DMA,
        ],
    )(xs, rows, cols_plain, cols)
    return out


def kernel(user_emb, item_emb, edge_index):
    n_users = user_emb.shape[0]
    n_items = item_emb.shape[0]
    ego = jnp.concatenate([user_emb, item_emb], axis=0)
    ego = jnp.pad(ego, ((0, _N_PAD - _N_REAL), (0, 0)))
    xs = ego.reshape(_N_PAD, _NC, _H).transpose(1, 0, 2)

    n_edges = edge_index.shape[1]
    pad = _E_PAD - n_edges
    dummy = jnp.full((pad,), _N_REAL, jnp.int32)
    rows = jnp.concatenate([edge_index[0], dummy]).reshape(_IDX_BLKS, _STREAM)
    cols_plain = jnp.concatenate([edge_index[1], dummy]).reshape(
        _IDX_BLKS, _STREAM)
    cols = jnp.stack([cols_plain, cols_plain + _N_PAD])

    out = _lightgcn(xs, rows, cols_plain, cols)
    full = out.transpose(1, 0, 2).reshape(_N_PAD, _NC * _H)
    return (full[:n_users], full[n_users:n_users + n_items])


# speedup vs baseline: 1.4973x; 1.0098x over previous
import jax
import jax.numpy as jnp
from jax import lax
from jax.experimental import pallas as pl
from jax.experimental.pallas import tpu as pltpu
from jax.experimental.pallas import tpu_sc as plsc

_N_REAL = 50000
_N_PAD = 51200
_H = 16
_NC = 2
_NS = 16
_ROWS_PER_TILE = _N_PAD // _NS
_RCHUNK = 320
_NCHUNK = _ROWS_PER_TILE // _RCHUNK
_E_PAD = 1_638_400
_STREAM = 1024
_IDX_BLKS = _E_PAD // _STREAM
_GROUPS = 4
_SB_PER_TILE = _IDX_BLKS // _NS // _GROUPS
_NG = _SB_PER_TILE * _GROUPS
_EPS = 1e-07
_N_LAYERS = 3


def _rsqrt_newton(a):
    i = lax.bitcast_convert_type(a, jnp.int32)
    i = jnp.int32(0x5F3759DF) - lax.shift_right_arithmetic(i, jnp.int32(1))
    r = lax.bitcast_convert_type(i, jnp.float32)
    half = a * 0.5
    for _ in range(3):
        r = r * (1.5 - half * r * r)
    return r


def _body(xs, rows, cols_plain, cols, out, y_hbm, c2_hbm, acc_sh,
          ribuf, cibuf, gring, abuf, gbuf, c2buf, sbuf, zerosb,
          gsem, ssem, isem):
    ci = lax.axis_index("c")
    tid = lax.axis_index("s")
    node_base = tid * _ROWS_PER_TILE
    blk_base = tid * _NG

    ones16 = jnp.ones((16,), jnp.float32)
    zeros16 = jnp.zeros((16,), jnp.float32)

    onesb = gring.at[0]

    def fill_ones(i, _):
        onesb[i, :] = ones16
        return 0

    def fill_zeros(i, _):
        zerosb[i, :] = zeros16
        return 0

    lax.fori_loop(0, _STREAM, fill_ones, 0)
    lax.fori_loop(0, _RCHUNK, fill_zeros, 0)

    def zero_chunk(c, _):
        off = node_base + c * _RCHUNK
        pltpu.sync_copy(zerosb, acc_sh.at[pl.ds(off, _RCHUNK)])
        return 0

    lax.fori_loop(0, _NCHUNK, zero_chunk, 0)
    plsc.subcore_barrier()

    def dscat_g(g, buf):
        ib = lax.rem(g // _GROUPS, 2)
        return (onesb, acc_sh.at[buf.at[ib, lax.rem(g, _GROUPS)]], ssem)

    def dfire_idx(sb):
        ib = lax.rem(sb, 2)
        blk_off = blk_base + sb * _GROUPS
        pltpu.async_copy(rows.at[pl.ds(blk_off, _GROUPS)], ribuf.at[ib],
                         isem)
        pltpu.async_copy(cols_plain.at[pl.ds(blk_off, _GROUPS)],
                         cibuf.at[ib], isem)

    def dwait_idx(sb):
        ib = lax.rem(sb, 2)
        blk_off = blk_base + sb * _GROUPS
        pltpu.make_async_copy(rows.at[pl.ds(blk_off, _GROUPS)],
                              ribuf.at[ib], isem).wait()
        pltpu.make_async_copy(cols_plain.at[pl.ds(blk_off, _GROUPS)],
                              cibuf.at[ib], isem).wait()

    dfire_idx(jnp.int32(0))
    dwait_idx(jnp.int32(0))
    dfire_idx(jnp.int32(1))

    def deg_body(g, _):
        @pl.when(g > 0)
        def _():
            pltpu.make_async_copy(*dscat_g(g - 1, ribuf)).wait()
            pltpu.make_async_copy(*dscat_g(g - 1, cibuf)).wait()

        @pl.when((lax.rem(g, _GROUPS) == 0) & (g > 0) & (g + _GROUPS < _NG))
        def _():
            dfire_idx(g // _GROUPS + 1)

        pltpu.async_copy(*dscat_g(g, ribuf), add=True)
        pltpu.async_copy(*dscat_g(g, cibuf), add=True)

        @pl.when((lax.rem(g, _GROUPS) == _GROUPS - 1) & (g + 1 < _NG))
        def _():
            dwait_idx((g + 1) // _GROUPS)

        return 0

    lax.fori_loop(0, _NG, deg_body, 0)
    pltpu.make_async_copy(*dscat_g(jnp.int32(_NG - 1), ribuf)).wait()
    pltpu.make_async_copy(*dscat_g(jnp.int32(_NG - 1), cibuf)).wait()
    plsc.subcore_barrier()

    def init_chunk(c, _):
        off = node_base + c * _RCHUNK
        pltpu.sync_copy(acc_sh.at[pl.ds(off, _RCHUNK)], abuf)
        pltpu.sync_copy(xs.at[ci, pl.ds(off, _RCHUNK)], gbuf)
        pltpu.sync_copy(zerosb, acc_sh.at[pl.ds(off, _RCHUNK)])

        def init_row(r, _):
            d = abuf[r, :] + _EPS
            c2 = 1.0 / d
            cc = c2 * _rsqrt_newton(c2)
            y0 = gbuf[r, :] * cc
            c2buf[r, :] = c2
            sbuf[r, :] = y0
            abuf[r, :] = y0
            return 0

        lax.fori_loop(0, _RCHUNK, init_row, 0)
        pltpu.sync_copy(c2buf, c2_hbm.at[ci, pl.ds(off, _RCHUNK)])
        pltpu.sync_copy(sbuf, out.at[ci, pl.ds(off, _RCHUNK)])
        pltpu.sync_copy(abuf, y_hbm.at[pl.ds(ci * _N_PAD + off, _RCHUNK)])
        return 0

    lax.fori_loop(0, _NCHUNK, init_chunk, 0)
    plsc.subcore_barrier()

    def gather_g(g):
        ib = lax.rem(g // _GROUPS, 2)
        return (y_hbm.at[cibuf.at[ib, lax.rem(g, _GROUPS)]],
                gring.at[lax.rem(g, 2)], gsem)

    def scatter_g(g):
        ib = lax.rem(g // _GROUPS, 2)
        return (gring.at[lax.rem(g, 2)],
                acc_sh.at[ribuf.at[ib, lax.rem(g, _GROUPS)]], ssem)

    def fire_idx(sb):
        ib = lax.rem(sb, 2)
        blk_off = blk_base + sb * _GROUPS
        pltpu.async_copy(rows.at[pl.ds(blk_off, _GROUPS)], ribuf.at[ib],
                         isem)
        pltpu.async_copy(cols.at[ci, pl.ds(blk_off, _GROUPS)], cibuf.at[ib],
                         isem)

    def wait_idx(sb):
        ib = lax.rem(sb, 2)
        blk_off = blk_base + sb * _GROUPS
        pltpu.make_async_copy(rows.at[pl.ds(blk_off, _GROUPS)],
                              ribuf.at[ib], isem).wait()
        pltpu.make_async_copy(cols.at[ci, pl.ds(blk_off, _GROUPS)],
                              cibuf.at[ib], isem).wait()

    def edge_pass():
        fire_idx(jnp.int32(0))
        wait_idx(jnp.int32(0))
        fire_idx(jnp.int32(1))
        pltpu.async_copy(*gather_g(jnp.int32(0)))

        def g_body(g, _):
            pltpu.make_async_copy(*gather_g(g)).wait()

            @pl.when(g > 0)
            def _():
                pltpu.make_async_copy(*scatter_g(g - 1)).wait()

            pltpu.async_copy(*scatter_g(g), add=True)

            @pl.when((lax.rem(g, _GROUPS) == 0) & (g > 0)
                     & (g + _GROUPS < _NG))
            def _():
                fire_idx(g // _GROUPS + 1)

            @pl.when(g + 1 < _NG)
            def _():
                @pl.when(lax.rem(g + 1, _GROUPS) == 0)
                def _():
                    wait_idx((g + 1) // _GROUPS)

                pltpu.async_copy(*gather_g(g + 1))

            return 0

        lax.fori_loop(0, _NG, g_body, 0)
        pltpu.make_async_copy(*scatter_g(jnp.int32(_NG - 1))).wait()

    for layer in range(_N_LAYERS):
        last = layer == _N_LAYERS - 1

        edge_pass()
        plsc.subcore_barrier()

        def rescale_chunk(c, _):
            off = node_base + c * _RCHUNK
            pltpu.sync_copy(acc_sh.at[pl.ds(off, _RCHUNK)], abuf)
            pltpu.sync_copy(c2_hbm.at[ci, pl.ds(off, _RCHUNK)], c2buf)
            pltpu.sync_copy(out.at[ci, pl.ds(off, _RCHUNK)], sbuf)
            if not last:
                pltpu.sync_copy(zerosb, acc_sh.at[pl.ds(off, _RCHUNK)])

            def rescale_row(r, _):
                c2 = c2buf[r, :]
                val = abuf[r, :] * c2
                s = sbuf[r, :] + val
                if last:
                    sbuf[r, :] = s * _rsqrt_newton(c2) * 0.25
                else:
                    sbuf[r, :] = s
                    abuf[r, :] = val
                return 0

            lax.fori_loop(0, _RCHUNK, rescale_row, 0)
            pltpu.sync_copy(sbuf, out.at[ci, pl.ds(off, _RCHUNK)])
            if not last:
                pltpu.sync_copy(
                    abuf, y_hbm.at[pl.ds(ci * _N_PAD + off, _RCHUNK)])
            return 0

        lax.fori_loop(0, _NCHUNK, rescale_chunk, 0)
        if not last:
            plsc.subcore_barrier()


@jax.jit
def _lightgcn(xs, rows, cols_plain, cols):
    mesh = plsc.VectorSubcoreMesh(core_axis_name="c", subcore_axis_name="s")
    out, _, _ = pl.kernel(
        _body,
        out_type=(
            jax.ShapeDtypeStruct((_NC, _N_PAD, _H), jnp.float32),
            jax.ShapeDtypeStruct((_NC * _N_PAD, _H), jnp.float32),
            jax.ShapeDtypeStruct((_NC, _N_PAD, _H), jnp.float32),
        ),
        mesh=mesh,
        compiler_params=pltpu.CompilerParams(use_tc_tiling_on_sc=False),
        scratch_types=[
            pltpu.VMEM_SHARED((_N_PAD, _H), jnp.float32),
            pltpu.VMEM((2, _GROUPS, _STREAM), jnp.int32),
            pltpu.VMEM((2, _GROUPS, _STREAM), jnp.int32),
            pltpu.VMEM((2, _STREAM, _H), jnp.float32),
            pltpu.VMEM((_RCHUNK, _H), jnp.float32),
            pltpu.VMEM((_RCHUNK, _H), jnp.float32),
            pltpu.VMEM((_RCHUNK, _H), jnp.float32),
            pltpu.VMEM((_RCHUNK, _H), jnp.float32),
            pltpu.VMEM((_RCHUNK, _H), jnp.float32),
            pltpu.SemaphoreType.DMA,
            pltpu.SemaphoreType.DMA,
            pltpu.SemaphoreType.DMA,
        ],
    )(xs, rows, cols_plain, cols)
    return out


def kernel(user_emb, item_emb, edge_index):
    n_users = user_emb.shape[0]
    n_items = item_emb.shape[0]
    ego = jnp.concatenate([user_emb, item_emb], axis=0)
    ego = jnp.pad(ego, ((0, _N_PAD - _N_REAL), (0, 0)))
    xs = ego.reshape(_N_PAD, _NC, _H).transpose(1, 0, 2)

    n_edges = edge_index.shape[1]
    pad = _E_PAD - n_edges
    dummy = jnp.full((pad,), _N_REAL, jnp.int32)
    rows = jnp.concatenate([edge_index[0], dummy]).reshape(_IDX_BLKS, _STREAM)
    cols_plain = jnp.concatenate([edge_index[1], dummy]).reshape(
        _IDX_BLKS, _STREAM)
    cols = jnp.stack([cols_plain, cols_plain + _N_PAD])

    out = _lightgcn(xs, rows, cols_plain, cols)
    full = out.transpose(1, 0, 2).reshape(_N_PAD, _NC * _H)
    return (full[:n_users], full[n_users:n_users + n_items])


# speedup vs baseline: 1.5624x; 1.0435x over previous
import jax
import jax.numpy as jnp
from jax import lax
from jax.experimental import pallas as pl
from jax.experimental.pallas import tpu as pltpu
from jax.experimental.pallas import tpu_sc as plsc

_N_REAL = 50000
_N_PAD = 51200
_H = 16
_NC = 2
_NS = 16
_ROWS_PER_TILE = _N_PAD // _NS
_RCHUNK = 320
_NCHUNK = _ROWS_PER_TILE // _RCHUNK
_E_PAD = 1_638_400
_STREAM = 256
_IDX_BLKS = _E_PAD // _STREAM
_GROUPS = 16
_SB_PER_TILE = _IDX_BLKS // _NS // _GROUPS
_NG = _SB_PER_TILE * _GROUPS
_D = 8
_AHEAD = _D // 2
_EPS = 1e-07
_N_LAYERS = 3


def _rsqrt_newton(a):
    i = lax.bitcast_convert_type(a, jnp.int32)
    i = jnp.int32(0x5F3759DF) - lax.shift_right_arithmetic(i, jnp.int32(1))
    r = lax.bitcast_convert_type(i, jnp.float32)
    half = a * 0.5
    for _ in range(3):
        r = r * (1.5 - half * r * r)
    return r


def _body(xs, rows, cols_plain, cols, out, y_hbm, c2_hbm, acc_sh,
          ribuf, cibuf, gring, abuf, gbuf, c2buf, sbuf, zerosb,
          gsem, ssem, isem):
    ci = lax.axis_index("c")
    tid = lax.axis_index("s")
    node_base = tid * _ROWS_PER_TILE
    blk_base = tid * _NG

    ones16 = jnp.ones((16,), jnp.float32)
    zeros16 = jnp.zeros((16,), jnp.float32)

    onesb = gring.at[0]

    def fill_ones(i, _):
        onesb[i, :] = ones16
        return 0

    def fill_zeros(i, _):
        zerosb[i, :] = zeros16
        return 0

    lax.fori_loop(0, _STREAM, fill_ones, 0)
    lax.fori_loop(0, _RCHUNK, fill_zeros, 0)

    def zero_chunk(c, _):
        off = node_base + c * _RCHUNK
        pltpu.sync_copy(zerosb, acc_sh.at[pl.ds(off, _RCHUNK)])
        return 0

    lax.fori_loop(0, _NCHUNK, zero_chunk, 0)
    plsc.subcore_barrier()

    def dscat_g(g, buf):
        ib = lax.rem(g // _GROUPS, 2)
        return (onesb, acc_sh.at[buf.at[ib, lax.rem(g, _GROUPS)]], ssem)

    def dfire_idx(sb):
        ib = lax.rem(sb, 2)
        blk_off = blk_base + sb * _GROUPS
        pltpu.async_copy(rows.at[pl.ds(blk_off, _GROUPS)], ribuf.at[ib],
                         isem)
        pltpu.async_copy(cols_plain.at[pl.ds(blk_off, _GROUPS)],
                         cibuf.at[ib], isem)

    def dwait_idx(sb):
        ib = lax.rem(sb, 2)
        blk_off = blk_base + sb * _GROUPS
        pltpu.make_async_copy(rows.at[pl.ds(blk_off, _GROUPS)],
                              ribuf.at[ib], isem).wait()
        pltpu.make_async_copy(cols_plain.at[pl.ds(blk_off, _GROUPS)],
                              cibuf.at[ib], isem).wait()

    dfire_idx(jnp.int32(0))
    dwait_idx(jnp.int32(0))
    dfire_idx(jnp.int32(1))

    def deg_body(g, _):
        @pl.when(g >= _AHEAD)
        def _():
            pltpu.make_async_copy(*dscat_g(g - _AHEAD, ribuf)).wait()
            pltpu.make_async_copy(*dscat_g(g - _AHEAD, cibuf)).wait()

        @pl.when((lax.rem(g, _GROUPS) == _AHEAD - 1) & (g > _GROUPS)
                 & (g + _GROUPS < _NG))
        def _():
            dfire_idx(g // _GROUPS + 1)

        pltpu.async_copy(*dscat_g(g, ribuf), add=True)
        pltpu.async_copy(*dscat_g(g, cibuf), add=True)

        @pl.when((lax.rem(g, _GROUPS) == _GROUPS - 1) & (g + 1 < _NG))
        def _():
            dwait_idx((g + 1) // _GROUPS)

        return 0

    lax.fori_loop(0, _NG, deg_body, 0)

    def deg_tail(t, _):
        g = _NG - _AHEAD + t
        pltpu.make_async_copy(*dscat_g(g, ribuf)).wait()
        pltpu.make_async_copy(*dscat_g(g, cibuf)).wait()
        return 0

    lax.fori_loop(0, _AHEAD, deg_tail, 0)
    plsc.subcore_barrier()

    def init_chunk(c, _):
        off = node_base + c * _RCHUNK
        pltpu.sync_copy(acc_sh.at[pl.ds(off, _RCHUNK)], abuf)
        pltpu.sync_copy(xs.at[ci, pl.ds(off, _RCHUNK)], gbuf)
        pltpu.sync_copy(zerosb, acc_sh.at[pl.ds(off, _RCHUNK)])

        def init_row(r, _):
            d = abuf[r, :] + _EPS
            c2 = 1.0 / d
            cc = c2 * _rsqrt_newton(c2)
            y0 = gbuf[r, :] * cc
            c2buf[r, :] = c2
            sbuf[r, :] = y0
            abuf[r, :] = y0
            return 0

        lax.fori_loop(0, _RCHUNK, init_row, 0)
        pltpu.sync_copy(c2buf, c2_hbm.at[ci, pl.ds(off, _RCHUNK)])
        pltpu.sync_copy(sbuf, out.at[ci, pl.ds(off, _RCHUNK)])
        pltpu.sync_copy(abuf, y_hbm.at[pl.ds(ci * _N_PAD + off, _RCHUNK)])
        return 0

    lax.fori_loop(0, _NCHUNK, init_chunk, 0)
    plsc.subcore_barrier()

    def gather_g(g):
        ib = lax.rem(g // _GROUPS, 2)
        return (y_hbm.at[cibuf.at[ib, lax.rem(g, _GROUPS)]],
                gring.at[lax.rem(g, _D)], gsem)

    def scatter_g(g):
        ib = lax.rem(g // _GROUPS, 2)
        return (gring.at[lax.rem(g, _D)],
                acc_sh.at[ribuf.at[ib, lax.rem(g, _GROUPS)]], ssem)

    def fire_idx(sb):
        ib = lax.rem(sb, 2)
        blk_off = blk_base + sb * _GROUPS
        pltpu.async_copy(rows.at[pl.ds(blk_off, _GROUPS)], ribuf.at[ib],
                         isem)
        pltpu.async_copy(cols.at[ci, pl.ds(blk_off, _GROUPS)], cibuf.at[ib],
                         isem)

    def wait_idx(sb):
        ib = lax.rem(sb, 2)
        blk_off = blk_base + sb * _GROUPS
        pltpu.make_async_copy(rows.at[pl.ds(blk_off, _GROUPS)],
                              ribuf.at[ib], isem).wait()
        pltpu.make_async_copy(cols.at[ci, pl.ds(blk_off, _GROUPS)],
                              cibuf.at[ib], isem).wait()

    def edge_pass():
        fire_idx(jnp.int32(0))
        wait_idx(jnp.int32(0))
        fire_idx(jnp.int32(1))
        for p in range(_AHEAD):
            pltpu.async_copy(*gather_g(jnp.int32(p)))

        def g_body(g, _):
            pltpu.make_async_copy(*gather_g(g)).wait()

            @pl.when(g >= _AHEAD)
            def _():
                pltpu.make_async_copy(*scatter_g(g - _AHEAD)).wait()

            pltpu.async_copy(*scatter_g(g), add=True)

            @pl.when((lax.rem(g, _GROUPS) == _AHEAD - 1) & (g > _GROUPS)
                     & (g + _GROUPS < _NG))
            def _():
                fire_idx(g // _GROUPS + 1)

            @pl.when(g + _AHEAD < _NG)
            def _():
                @pl.when(lax.rem(g + _AHEAD, _GROUPS) == 0)
                def _():
                    wait_idx((g + _AHEAD) // _GROUPS)

                pltpu.async_copy(*gather_g(g + _AHEAD))

            return 0

        lax.fori_loop(0, _NG, g_body, 0)

        def s_tail(t, _):
            pltpu.make_async_copy(*scatter_g(_NG - _AHEAD + t)).wait()
            return 0

        lax.fori_loop(0, _AHEAD, s_tail, 0)

    for layer in range(_N_LAYERS):
        last = layer == _N_LAYERS - 1

        edge_pass()
        plsc.subcore_barrier()

        def rescale_chunk(c, _):
            off = node_base + c * _RCHUNK
            pltpu.sync_copy(acc_sh.at[pl.ds(off, _RCHUNK)], abuf)
            pltpu.sync_copy(c2_hbm.at[ci, pl.ds(off, _RCHUNK)], c2buf)
            pltpu.sync_copy(out.at[ci, pl.ds(off, _RCHUNK)], sbuf)
            if not last:
                pltpu.sync_copy(zerosb, acc_sh.at[pl.ds(off, _RCHUNK)])

            def rescale_row(r, _):
                c2 = c2buf[r, :]
                val = abuf[r, :] * c2
                s = sbuf[r, :] + val
                if last:
                    sbuf[r, :] = s * _rsqrt_newton(c2) * 0.25
                else:
                    sbuf[r, :] = s
                    abuf[r, :] = val
                return 0

            lax.fori_loop(0, _RCHUNK, rescale_row, 0)
            pltpu.sync_copy(sbuf, out.at[ci, pl.ds(off, _RCHUNK)])
            if not last:
                pltpu.sync_copy(
                    abuf, y_hbm.at[pl.ds(ci * _N_PAD + off, _RCHUNK)])
            return 0

        lax.fori_loop(0, _NCHUNK, rescale_chunk, 0)
        if not last:
            plsc.subcore_barrier()


@jax.jit
def _lightgcn(xs, rows, cols_plain, cols):
    mesh = plsc.VectorSubcoreMesh(core_axis_name="c", subcore_axis_name="s")
    out, _, _ = pl.kernel(
        _body,
        out_type=(
            jax.ShapeDtypeStruct((_NC, _N_PAD, _H), jnp.float32),
            jax.ShapeDtypeStruct((_NC * _N_PAD, _H), jnp.float32),
            jax.ShapeDtypeStruct((_NC, _N_PAD, _H), jnp.float32),
        ),
        mesh=mesh,
        compiler_params=pltpu.CompilerParams(use_tc_tiling_on_sc=False),
        scratch_types=[
            pltpu.VMEM_SHARED((_N_PAD, _H), jnp.float32),
            pltpu.VMEM((2, _GROUPS, _STREAM), jnp.int32),
            pltpu.VMEM((2, _GROUPS, _STREAM), jnp.int32),
            pltpu.VMEM((_D, _STREAM, _H), jnp.float32),
            pltpu.VMEM((_RCHUNK, _H), jnp.float32),
            pltpu.VMEM((_RCHUNK, _H), jnp.float32),
            pltpu.VMEM((_RCHUNK, _H), jnp.float32),
            pltpu.VMEM((_RCHUNK, _H), jnp.float32),
            pltpu.VMEM((_RCHUNK, _H), jnp.float32),
            pltpu.SemaphoreType.DMA,
            pltpu.SemaphoreType.DMA,
            pltpu.SemaphoreType.DMA,
        ],
    )(xs, rows, cols_plain, cols)
    return out


def kernel(user_emb, item_emb, edge_index):
    n_users = user_emb.shape[0]
    n_items = item_emb.shape[0]
    ego = jnp.concatenate([user_emb, item_emb], axis=0)
    ego = jnp.pad(ego, ((0, _N_PAD - _N_REAL), (0, 0)))
    xs = ego.reshape(_N_PAD, _NC, _H).transpose(1, 0, 2)

    n_edges = edge_index.shape[1]
    pad = _E_PAD - n_edges
    dummy = jnp.full((pad,), _N_REAL, jnp.int32)
    rows = jnp.concatenate([edge_index[0], dummy]).reshape(_IDX_BLKS, _STREAM)
    cols_plain = jnp.concatenate([edge_index[1], dummy]).reshape(
        _IDX_BLKS, _STREAM)
    cols = jnp.stack([cols_plain, cols_plain + _N_PAD])

    out = _lightgcn(xs, rows, cols_plain, cols)
    full = out.transpose(1, 0, 2).reshape(_N_PAD, _NC * _H)
    return (full[:n_users], full[n_users:n_users + n_items])


# speedup vs baseline: 1.6331x; 1.0452x over previous
import jax
import jax.numpy as jnp
from jax import lax
from jax.experimental import pallas as pl
from jax.experimental.pallas import tpu as pltpu
from jax.experimental.pallas import tpu_sc as plsc

_N_REAL = 50000
_N_PAD = 51200
_H = 16
_NC = 2
_NS = 16
_ROWS_PER_TILE = _N_PAD // _NS
_RCHUNK = 320
_NCHUNK = _ROWS_PER_TILE // _RCHUNK
_E_PAD = 1_638_400
_STREAM = 128
_IDX_BLKS = _E_PAD // _STREAM
_GROUPS = 32
_SB_PER_TILE = _IDX_BLKS // _NS // _GROUPS
_NG = _SB_PER_TILE * _GROUPS
_D = 16
_AHEAD = _D // 2
_EPS = 1e-07
_N_LAYERS = 3


def _rsqrt_newton(a):
    i = lax.bitcast_convert_type(a, jnp.int32)
    i = jnp.int32(0x5F3759DF) - lax.shift_right_arithmetic(i, jnp.int32(1))
    r = lax.bitcast_convert_type(i, jnp.float32)
    half = a * 0.5
    for _ in range(3):
        r = r * (1.5 - half * r * r)
    return r


def _body(xs, rows, cols_plain, cols, out, y_hbm, c2_hbm, acc_sh,
          ribuf, cibuf, gring, abuf, gbuf, c2buf, sbuf, zerosb,
          gsem, ssem, isem):
    ci = lax.axis_index("c")
    tid = lax.axis_index("s")
    node_base = tid * _ROWS_PER_TILE
    blk_base = tid * _NG

    ones16 = jnp.ones((16,), jnp.float32)
    zeros16 = jnp.zeros((16,), jnp.float32)

    onesb = gring.at[0]

    def fill_ones(i, _):
        onesb[i, :] = ones16
        return 0

    def fill_zeros(i, _):
        zerosb[i, :] = zeros16
        return 0

    lax.fori_loop(0, _STREAM, fill_ones, 0)
    lax.fori_loop(0, _RCHUNK, fill_zeros, 0)

    def zero_chunk(c, _):
        off = node_base + c * _RCHUNK
        pltpu.sync_copy(zerosb, acc_sh.at[pl.ds(off, _RCHUNK)])
        return 0

    lax.fori_loop(0, _NCHUNK, zero_chunk, 0)
    plsc.subcore_barrier()

    def dscat_g(g, buf):
        ib = lax.rem(g // _GROUPS, 2)
        return (onesb, acc_sh.at[buf.at[ib, lax.rem(g, _GROUPS)]], ssem)

    def dfire_idx(sb):
        ib = lax.rem(sb, 2)
        blk_off = blk_base + sb * _GROUPS
        pltpu.async_copy(rows.at[pl.ds(blk_off, _GROUPS)], ribuf.at[ib],
                         isem)
        pltpu.async_copy(cols_plain.at[pl.ds(blk_off, _GROUPS)],
                         cibuf.at[ib], isem)

    def dwait_idx(sb):
        ib = lax.rem(sb, 2)
        blk_off = blk_base + sb * _GROUPS
        pltpu.make_async_copy(rows.at[pl.ds(blk_off, _GROUPS)],
                              ribuf.at[ib], isem).wait()
        pltpu.make_async_copy(cols_plain.at[pl.ds(blk_off, _GROUPS)],
                              cibuf.at[ib], isem).wait()

    dfire_idx(jnp.int32(0))
    dwait_idx(jnp.int32(0))
    dfire_idx(jnp.int32(1))

    def deg_body(g, _):
        @pl.when(g >= _AHEAD)
        def _():
            pltpu.make_async_copy(*dscat_g(g - _AHEAD, ribuf)).wait()
            pltpu.make_async_copy(*dscat_g(g - _AHEAD, cibuf)).wait()

        @pl.when((lax.rem(g, _GROUPS) == _AHEAD - 1) & (g > _GROUPS)
                 & (g + _GROUPS < _NG))
        def _():
            dfire_idx(g // _GROUPS + 1)

        pltpu.async_copy(*dscat_g(g, ribuf), add=True)
        pltpu.async_copy(*dscat_g(g, cibuf), add=True)

        @pl.when((lax.rem(g, _GROUPS) == _GROUPS - 1) & (g + 1 < _NG))
        def _():
            dwait_idx((g + 1) // _GROUPS)

        return 0

    lax.fori_loop(0, _NG, deg_body, 0)

    def deg_tail(t, _):
        g = _NG - _AHEAD + t
        pltpu.make_async_copy(*dscat_g(g, ribuf)).wait()
        pltpu.make_async_copy(*dscat_g(g, cibuf)).wait()
        return 0

    lax.fori_loop(0, _AHEAD, deg_tail, 0)
    plsc.subcore_barrier()

    def init_chunk(c, _):
        off = node_base + c * _RCHUNK
        pltpu.sync_copy(acc_sh.at[pl.ds(off, _RCHUNK)], abuf)
        pltpu.sync_copy(xs.at[ci, pl.ds(off, _RCHUNK)], gbuf)
        pltpu.sync_copy(zerosb, acc_sh.at[pl.ds(off, _RCHUNK)])

        def init_row(r, _):
            d = abuf[r, :] + _EPS
            c2 = 1.0 / d
            cc = c2 * _rsqrt_newton(c2)
            y0 = gbuf[r, :] * cc
            c2buf[r, :] = c2
            sbuf[r, :] = y0
            abuf[r, :] = y0
            return 0

        lax.fori_loop(0, _RCHUNK, init_row, 0)
        pltpu.sync_copy(c2buf, c2_hbm.at[ci, pl.ds(off, _RCHUNK)])
        pltpu.sync_copy(sbuf, out.at[ci, pl.ds(off, _RCHUNK)])
        pltpu.sync_copy(abuf, y_hbm.at[pl.ds(ci * _N_PAD + off, _RCHUNK)])
        return 0

    lax.fori_loop(0, _NCHUNK, init_chunk, 0)
    plsc.subcore_barrier()

    def gather_g(g):
        ib = lax.rem(g // _GROUPS, 2)
        return (y_hbm.at[cibuf.at[ib, lax.rem(g, _GROUPS)]],
                gring.at[lax.rem(g, _D)], gsem)

    def scatter_g(g):
        ib = lax.rem(g // _GROUPS, 2)
        return (gring.at[lax.rem(g, _D)],
                acc_sh.at[ribuf.at[ib, lax.rem(g, _GROUPS)]], ssem)

    def fire_idx(sb):
        ib = lax.rem(sb, 2)
        blk_off = blk_base + sb * _GROUPS
        pltpu.async_copy(rows.at[pl.ds(blk_off, _GROUPS)], ribuf.at[ib],
                         isem)
        pltpu.async_copy(cols.at[ci, pl.ds(blk_off, _GROUPS)], cibuf.at[ib],
                         isem)

    def wait_idx(sb):
        ib = lax.rem(sb, 2)
        blk_off = blk_base + sb * _GROUPS
        pltpu.make_async_copy(rows.at[pl.ds(blk_off, _GROUPS)],
                              ribuf.at[ib], isem).wait()
        pltpu.make_async_copy(cols.at[ci, pl.ds(blk_off, _GROUPS)],
                              cibuf.at[ib], isem).wait()

    def edge_pass():
        fire_idx(jnp.int32(0))
        wait_idx(jnp.int32(0))
        fire_idx(jnp.int32(1))
        for p in range(_AHEAD):
            pltpu.async_copy(*gather_g(jnp.int32(p)))

        def g_body(g, _):
            pltpu.make_async_copy(*gather_g(g)).wait()

            @pl.when(g >= _AHEAD)
            def _():
                pltpu.make_async_copy(*scatter_g(g - _AHEAD)).wait()

            pltpu.async_copy(*scatter_g(g), add=True)

            @pl.when((lax.rem(g, _GROUPS) == _AHEAD - 1) & (g > _GROUPS)
                     & (g + _GROUPS < _NG))
            def _():
                fire_idx(g // _GROUPS + 1)

            @pl.when(g + _AHEAD < _NG)
            def _():
                @pl.when(lax.rem(g + _AHEAD, _GROUPS) == 0)
                def _():
                    wait_idx((g + _AHEAD) // _GROUPS)

                pltpu.async_copy(*gather_g(g + _AHEAD))

            return 0

        lax.fori_loop(0, _NG, g_body, 0)

        def s_tail(t, _):
            pltpu.make_async_copy(*scatter_g(_NG - _AHEAD + t)).wait()
            return 0

        lax.fori_loop(0, _AHEAD, s_tail, 0)

    for layer in range(_N_LAYERS):
        last = layer == _N_LAYERS - 1

        edge_pass()
        plsc.subcore_barrier()

        def rescale_chunk(c, _):
            off = node_base + c * _RCHUNK
            pltpu.sync_copy(acc_sh.at[pl.ds(off, _RCHUNK)], abuf)
            pltpu.sync_copy(c2_hbm.at[ci, pl.ds(off, _RCHUNK)], c2buf)
            pltpu.sync_copy(out.at[ci, pl.ds(off, _RCHUNK)], sbuf)
            if not last:
                pltpu.sync_copy(zerosb, acc_sh.at[pl.ds(off, _RCHUNK)])

            def rescale_row(r, _):
                c2 = c2buf[r, :]
                val = abuf[r, :] * c2
                s = sbuf[r, :] + val
                if last:
                    sbuf[r, :] = s * _rsqrt_newton(c2) * 0.25
                else:
                    sbuf[r, :] = s
                    abuf[r, :] = val
                return 0

            lax.fori_loop(0, _RCHUNK, rescale_row, 0)
            pltpu.sync_copy(sbuf, out.at[ci, pl.ds(off, _RCHUNK)])
            if not last:
                pltpu.sync_copy(
                    abuf, y_hbm.at[pl.ds(ci * _N_PAD + off, _RCHUNK)])
            return 0

        lax.fori_loop(0, _NCHUNK, rescale_chunk, 0)
        if not last:
            plsc.subcore_barrier()


@jax.jit
def _lightgcn(xs, rows, cols_plain, cols):
    mesh = plsc.VectorSubcoreMesh(core_axis_name="c", subcore_axis_name="s")
    out, _, _ = pl.kernel(
        _body,
        out_type=(
            jax.ShapeDtypeStruct((_NC, _N_PAD, _H), jnp.float32),
            jax.ShapeDtypeStruct((_NC * _N_PAD, _H), jnp.float32),
            jax.ShapeDtypeStruct((_NC, _N_PAD, _H), jnp.float32),
        ),
        mesh=mesh,
        compiler_params=pltpu.CompilerParams(use_tc_tiling_on_sc=False),
        scratch_types=[
            pltpu.VMEM_SHARED((_N_PAD, _H), jnp.float32),
            pltpu.VMEM((2, _GROUPS, _STREAM), jnp.int32),
            pltpu.VMEM((2, _GROUPS, _STREAM), jnp.int32),
            pltpu.VMEM((_D, _STREAM, _H), jnp.float32),
            pltpu.VMEM((_RCHUNK, _H), jnp.float32),
            pltpu.VMEM((_RCHUNK, _H), jnp.float32),
            pltpu.VMEM((_RCHUNK, _H), jnp.float32),
            pltpu.VMEM((_RCHUNK, _H), jnp.float32),
            pltpu.VMEM((_RCHUNK, _H), jnp.float32),
            pltpu.SemaphoreType.DMA,
            pltpu.SemaphoreType.DMA,
            pltpu.SemaphoreType.DMA,
        ],
    )(xs, rows, cols_plain, cols)
    return out


def kernel(user_emb, item_emb, edge_index):
    n_users = user_emb.shape[0]
    n_items = item_emb.shape[0]
    ego = jnp.concatenate([user_emb, item_emb], axis=0)
    ego = jnp.pad(ego, ((0, _N_PAD - _N_REAL), (0, 0)))
    xs = ego.reshape(_N_PAD, _NC, _H).transpose(1, 0, 2)

    n_edges = edge_index.shape[1]
    pad = _E_PAD - n_edges
    dummy = jnp.full((pad,), _N_REAL, jnp.int32)
    rows = jnp.concatenate([edge_index[0], dummy]).reshape(_IDX_BLKS, _STREAM)
    cols_plain = jnp.concatenate([edge_index[1], dummy]).reshape(
        _IDX_BLKS, _STREAM)
    cols = jnp.stack([cols_plain, cols_plain + _N_PAD])

    out = _lightgcn(xs, rows, cols_plain, cols)
    full = out.transpose(1, 0, 2).reshape(_N_PAD, _NC * _H)
    return (full[:n_users], full[n_users:n_users + n_items])


# speedup vs baseline: 1.6803x; 1.0289x over previous
import jax
import jax.numpy as jnp
from jax import lax
from jax.experimental import pallas as pl
from jax.experimental.pallas import tpu as pltpu
from jax.experimental.pallas import tpu_sc as plsc

_N_REAL = 50000
_N_PAD = 51200
_H = 16
_NC = 2
_NS = 16
_ROWS_PER_TILE = _N_PAD // _NS
_RCHUNK = 320
_NCHUNK = _ROWS_PER_TILE // _RCHUNK
_E_PAD = 1_638_400
_STREAM = 64
_IDX_BLKS = _E_PAD // _STREAM
_GROUPS = 64
_SB_PER_TILE = _IDX_BLKS // _NS // _GROUPS
_NG = _SB_PER_TILE * _GROUPS
_D = 32
_AHEAD = _D // 2
_EPS = 1e-07
_N_LAYERS = 3


def _rsqrt_newton(a):
    i = lax.bitcast_convert_type(a, jnp.int32)
    i = jnp.int32(0x5F3759DF) - lax.shift_right_arithmetic(i, jnp.int32(1))
    r = lax.bitcast_convert_type(i, jnp.float32)
    half = a * 0.5
    for _ in range(3):
        r = r * (1.5 - half * r * r)
    return r


def _body(xs, rows, cols_plain, cols, out, y_hbm, c2_hbm, acc_sh,
          ribuf, cibuf, gring, abuf, gbuf, c2buf, sbuf, zerosb,
          gsem, ssem, isem):
    ci = lax.axis_index("c")
    tid = lax.axis_index("s")
    node_base = tid * _ROWS_PER_TILE
    blk_base = tid * _NG

    ones16 = jnp.ones((16,), jnp.float32)
    zeros16 = jnp.zeros((16,), jnp.float32)

    onesb = gring.at[0]

    def fill_ones(i, _):
        onesb[i, :] = ones16
        return 0

    def fill_zeros(i, _):
        zerosb[i, :] = zeros16
        return 0

    lax.fori_loop(0, _STREAM, fill_ones, 0)
    lax.fori_loop(0, _RCHUNK, fill_zeros, 0)

    def zero_chunk(c, _):
        off = node_base + c * _RCHUNK
        pltpu.sync_copy(zerosb, acc_sh.at[pl.ds(off, _RCHUNK)])
        return 0

    lax.fori_loop(0, _NCHUNK, zero_chunk, 0)
    plsc.subcore_barrier()

    def dscat_g(g, buf):
        ib = lax.rem(g // _GROUPS, 2)
        return (onesb, acc_sh.at[buf.at[ib, lax.rem(g, _GROUPS)]], ssem)

    def dfire_idx(sb):
        ib = lax.rem(sb, 2)
        blk_off = blk_base + sb * _GROUPS
        pltpu.async_copy(rows.at[pl.ds(blk_off, _GROUPS)], ribuf.at[ib],
                         isem)
        pltpu.async_copy(cols_plain.at[pl.ds(blk_off, _GROUPS)],
                         cibuf.at[ib], isem)

    def dwait_idx(sb):
        ib = lax.rem(sb, 2)
        blk_off = blk_base + sb * _GROUPS
        pltpu.make_async_copy(rows.at[pl.ds(blk_off, _GROUPS)],
                              ribuf.at[ib], isem).wait()
        pltpu.make_async_copy(cols_plain.at[pl.ds(blk_off, _GROUPS)],
                              cibuf.at[ib], isem).wait()

    dfire_idx(jnp.int32(0))
    dwait_idx(jnp.int32(0))
    dfire_idx(jnp.int32(1))

    def deg_body(g, _):
        @pl.when(g >= _AHEAD)
        def _():
            pltpu.make_async_copy(*dscat_g(g - _AHEAD, ribuf)).wait()
            pltpu.make_async_copy(*dscat_g(g - _AHEAD, cibuf)).wait()

        @pl.when((lax.rem(g, _GROUPS) == _AHEAD - 1) & (g > _GROUPS)
                 & (g + _GROUPS < _NG))
        def _():
            dfire_idx(g // _GROUPS + 1)

        pltpu.async_copy(*dscat_g(g, ribuf), add=True)
        pltpu.async_copy(*dscat_g(g, cibuf), add=True)

        @pl.when((lax.rem(g, _GROUPS) == _GROUPS - 1) & (g + 1 < _NG))
        def _():
            dwait_idx((g + 1) // _GROUPS)

        return 0

    lax.fori_loop(0, _NG, deg_body, 0)

    def deg_tail(t, _):
        g = _NG - _AHEAD + t
        pltpu.make_async_copy(*dscat_g(g, ribuf)).wait()
        pltpu.make_async_copy(*dscat_g(g, cibuf)).wait()
        return 0

    lax.fori_loop(0, _AHEAD, deg_tail, 0)
    plsc.subcore_barrier()

    def init_chunk(c, _):
        off = node_base + c * _RCHUNK
        pltpu.sync_copy(acc_sh.at[pl.ds(off, _RCHUNK)], abuf)
        pltpu.sync_copy(xs.at[ci, pl.ds(off, _RCHUNK)], gbuf)
        pltpu.sync_copy(zerosb, acc_sh.at[pl.ds(off, _RCHUNK)])

        def init_row(r, _):
            d = abuf[r, :] + _EPS
            c2 = 1.0 / d
            cc = c2 * _rsqrt_newton(c2)
            y0 = gbuf[r, :] * cc
            c2buf[r, :] = c2
            sbuf[r, :] = y0
            abuf[r, :] = y0
            return 0

        lax.fori_loop(0, _RCHUNK, init_row, 0)
        pltpu.sync_copy(c2buf, c2_hbm.at[ci, pl.ds(off, _RCHUNK)])
        pltpu.sync_copy(sbuf, out.at[ci, pl.ds(off, _RCHUNK)])
        pltpu.sync_copy(abuf, y_hbm.at[pl.ds(ci * _N_PAD + off, _RCHUNK)])
        return 0

    lax.fori_loop(0, _NCHUNK, init_chunk, 0)
    plsc.subcore_barrier()

    def gather_g(g):
        ib = lax.rem(g // _GROUPS, 2)
        return (y_hbm.at[cibuf.at[ib, lax.rem(g, _GROUPS)]],
                gring.at[lax.rem(g, _D)], gsem)

    def scatter_g(g):
        ib = lax.rem(g // _GROUPS, 2)
        return (gring.at[lax.rem(g, _D)],
                acc_sh.at[ribuf.at[ib, lax.rem(g, _GROUPS)]], ssem)

    def fire_idx(sb):
        ib = lax.rem(sb, 2)
        blk_off = blk_base + sb * _GROUPS
        pltpu.async_copy(rows.at[pl.ds(blk_off, _GROUPS)], ribuf.at[ib],
                         isem)
        pltpu.async_copy(cols.at[ci, pl.ds(blk_off, _GROUPS)], cibuf.at[ib],
                         isem)

    def wait_idx(sb):
        ib = lax.rem(sb, 2)
        blk_off = blk_base + sb * _GROUPS
        pltpu.make_async_copy(rows.at[pl.ds(blk_off, _GROUPS)],
                              ribuf.at[ib], isem).wait()
        pltpu.make_async_copy(cols.at[ci, pl.ds(blk_off, _GROUPS)],
                              cibuf.at[ib], isem).wait()

    def edge_pass():
        fire_idx(jnp.int32(0))
        wait_idx(jnp.int32(0))
        fire_idx(jnp.int32(1))
        for p in range(_AHEAD):
            pltpu.async_copy(*gather_g(jnp.int32(p)))

        def g_body(g, _):
            pltpu.make_async_copy(*gather_g(g)).wait()

            @pl.when(g >= _AHEAD)
            def _():
                pltpu.make_async_copy(*scatter_g(g - _AHEAD)).wait()

            pltpu.async_copy(*scatter_g(g), add=True)

            @pl.when((lax.rem(g, _GROUPS) == _AHEAD - 1) & (g > _GROUPS)
                     & (g + _GROUPS < _NG))
            def _():
                fire_idx(g // _GROUPS + 1)

            @pl.when(g + _AHEAD < _NG)
            def _():
                @pl.when(lax.rem(g + _AHEAD, _GROUPS) == 0)
                def _():
                    wait_idx((g + _AHEAD) // _GROUPS)

                pltpu.async_copy(*gather_g(g + _AHEAD))

            return 0

        lax.fori_loop(0, _NG, g_body, 0)

        def s_tail(t, _):
            pltpu.make_async_copy(*scatter_g(_NG - _AHEAD + t)).wait()
            return 0

        lax.fori_loop(0, _AHEAD, s_tail, 0)

    for layer in range(_N_LAYERS):
        last = layer == _N_LAYERS - 1

        edge_pass()
        plsc.subcore_barrier()

        def rescale_chunk(c, _):
            off = node_base + c * _RCHUNK
            pltpu.sync_copy(acc_sh.at[pl.ds(off, _RCHUNK)], abuf)
            pltpu.sync_copy(c2_hbm.at[ci, pl.ds(off, _RCHUNK)], c2buf)
            pltpu.sync_copy(out.at[ci, pl.ds(off, _RCHUNK)], sbuf)
            if not last:
                pltpu.sync_copy(zerosb, acc_sh.at[pl.ds(off, _RCHUNK)])

            def rescale_row(r, _):
                c2 = c2buf[r, :]
                val = abuf[r, :] * c2
                s = sbuf[r, :] + val
                if last:
                    sbuf[r, :] = s * _rsqrt_newton(c2) * 0.25
                else:
                    sbuf[r, :] = s
                    abuf[r, :] = val
                return 0

            lax.fori_loop(0, _RCHUNK, rescale_row, 0)
            pltpu.sync_copy(sbuf, out.at[ci, pl.ds(off, _RCHUNK)])
            if not last:
                pltpu.sync_copy(
                    abuf, y_hbm.at[pl.ds(ci * _N_PAD + off, _RCHUNK)])
            return 0

        lax.fori_loop(0, _NCHUNK, rescale_chunk, 0)
        if not last:
            plsc.subcore_barrier()


@jax.jit
def _lightgcn(xs, rows, cols_plain, cols):
    mesh = plsc.VectorSubcoreMesh(core_axis_name="c", subcore_axis_name="s")
    out, _, _ = pl.kernel(
        _body,
        out_type=(
            jax.ShapeDtypeStruct((_NC, _N_PAD, _H), jnp.float32),
            jax.ShapeDtypeStruct((_NC * _N_PAD, _H), jnp.float32),
            jax.ShapeDtypeStruct((_NC, _N_PAD, _H), jnp.float32),
        ),
        mesh=mesh,
        compiler_params=pltpu.CompilerParams(use_tc_tiling_on_sc=False),
        scratch_types=[
            pltpu.VMEM_SHARED((_N_PAD, _H), jnp.float32),
            pltpu.VMEM((2, _GROUPS, _STREAM), jnp.int32),
            pltpu.VMEM((2, _GROUPS, _STREAM), jnp.int32),
            pltpu.VMEM((_D, _STREAM, _H), jnp.float32),
            pltpu.VMEM((_RCHUNK, _H), jnp.float32),
            pltpu.VMEM((_RCHUNK, _H), jnp.float32),
            pltpu.VMEM((_RCHUNK, _H), jnp.float32),
            pltpu.VMEM((_RCHUNK, _H), jnp.float32),
            pltpu.VMEM((_RCHUNK, _H), jnp.float32),
            pltpu.SemaphoreType.DMA,
            pltpu.SemaphoreType.DMA,
            pltpu.SemaphoreType.DMA,
        ],
    )(xs, rows, cols_plain, cols)
    return out


def kernel(user_emb, item_emb, edge_index):
    n_users = user_emb.shape[0]
    n_items = item_emb.shape[0]
    ego = jnp.concatenate([user_emb, item_emb], axis=0)
    ego = jnp.pad(ego, ((0, _N_PAD - _N_REAL), (0, 0)))
    xs = ego.reshape(_N_PAD, _NC, _H).transpose(1, 0, 2)

    n_edges = edge_index.shape[1]
    pad = _E_PAD - n_edges
    dummy = jnp.full((pad,), _N_REAL, jnp.int32)
    rows = jnp.concatenate([edge_index[0], dummy]).reshape(_IDX_BLKS, _STREAM)
    cols_plain = jnp.concatenate([edge_index[1], dummy]).reshape(
        _IDX_BLKS, _STREAM)
    cols = jnp.stack([cols_plain, cols_plain + _N_PAD])

    out = _lightgcn(xs, rows, cols_plain, cols)
    full = out.transpose(1, 0, 2).reshape(_N_PAD, _NC * _H)
    return (full[:n_users], full[n_users:n_users + n_items])
